# baseline (speedup 1.0000x reference)
import functools

import jax
import jax.numpy as jnp
from jax import lax
from jax.experimental import pallas as pl
from jax.experimental.pallas import tpu as pltpu
from jax.experimental.pallas import tpu_sc as plsc

_INFO = plsc.get_sparse_core_info()
_NC = _INFO.num_cores
_NS = _INFO.num_subcores
_NW = _NC * _NS
_LANES = _INFO.num_lanes


def _make_sc_kernel(V, D, B, L):
    assert L % 2 == 0
    half = L // 2
    assert B % (8 * _NW) == 0 and D % _LANES == 0
    bpw = B // _NW
    ncol = D // _LANES

    mesh = plsc.VectorSubcoreMesh(core_axis_name="c", subcore_axis_name="s")

    @functools.partial(
        pl.kernel,
        mesh=mesh,
        out_type=jax.ShapeDtypeStruct((B, D), jnp.float32),
        scratch_types=[
            pltpu.VMEM((bpw, 2, half), jnp.int32),
            pltpu.VMEM((half, D), jnp.float32),
            pltpu.VMEM((half, D), jnp.float32),
            pltpu.VMEM((half, D), jnp.float32),
            pltpu.VMEM((half, D), jnp.float32),
            pltpu.VMEM((half, D), jnp.float32),
            pltpu.VMEM((half, D), jnp.float32),
            pltpu.VMEM((bpw, D), jnp.float32),
            pltpu.SemaphoreType.DMA,
            pltpu.SemaphoreType.DMA,
            pltpu.SemaphoreType.DMA,
            pltpu.SemaphoreType.DMA,
            pltpu.SemaphoreType.DMA,
            pltpu.SemaphoreType.DMA,
            pltpu.SemaphoreType.DMA,
        ],
    )
    def k(table_hbm, idx_hbm, out_hbm, idx_v,
          buf0, buf1, buf2, buf3, buf4, buf5, out_v,
          sem0, sem1, sem2, sem3, sem4, sem5, sem_idx):
        wid = lax.axis_index("s") * _NC + lax.axis_index("c")
        base = wid * bpw

        bufs = ((buf0, sem0), (buf1, sem1), (buf2, sem2),
                (buf3, sem3), (buf4, sem4), (buf5, sem5))
        nbuf = len(bufs)
        nprime = nbuf // 2

        pltpu.sync_copy(idx_hbm.at[pl.ds(base, nprime)],
                        idx_v.at[pl.ds(0, nprime)])
        idx_rest = pltpu.make_async_copy(
            idx_hbm.at[pl.ds(base + nprime, bpw - nprime)],
            idx_v.at[pl.ds(nprime, bpw - nprime)],
            sem_idx,
        )
        idx_rest.start()

        splits = ((0, 48), (48, half - 48))

        def start_half(i, h, buf, sem):
            for off, n in splits:
                pltpu.async_copy(
                    table_hbm.at[idx_v.at[i, h, pl.ds(off, n)]],
                    buf.at[pl.ds(off, n)],
                    sem,
                )

        def wait_half(i, h, buf, sem):
            pltpu.make_async_copy(
                table_hbm.at[idx_v.at[i, h]], buf, sem
            ).wait()

        UNROLL = 4
        assert half % UNROLL == 0

        acc_init = tuple(
            jnp.full((_LANES,), -jnp.inf, jnp.float32) for _ in range(ncol)
        )

        def compute_half(buf, acc_in):
            def row_body(jb, acc):
                j = jb * UNROLL
                out = []
                for c in range(ncol):
                    s = pl.ds(c * _LANES, _LANES)
                    m = [buf[j + u, s] for u in range(UNROLL)]
                    while len(m) > 1:
                        m = [jnp.maximum(m[2 * t], m[2 * t + 1])
                             for t in range(len(m) // 2)] + m[len(m) - len(m) % 2:]
                    out.append(jnp.maximum(acc[c], m[0]))
                return tuple(out)

            return lax.fori_loop(0, half // UNROLL, row_body, acc_in)

        def store(i, acc):
            for c in range(ncol):
                out_v[i, pl.ds(c * _LANES, _LANES)] = acc[c]

        for s in range(nbuf):
            start_half(s // 2, s % 2, bufs[s][0], bufs[s][1])
        idx_rest.wait()

        total_halves = 2 * bpw
        main_halves = (total_halves // nbuf) * nbuf
        assert nbuf % 2 == 0

        def loop_body(step, _):
            ebase = step * (nbuf // 2)
            acc = acc_init
            for b in range(nbuf):
                i = ebase + b // 2
                h = b % 2
                buf, sem = bufs[b]
                wait_half(i, h, buf, sem)
                acc = compute_half(buf, acc_init if h == 0 else acc)
                if h == 1:
                    store(i, acc)

                @pl.when(i + nbuf // 2 < bpw)
                def _():
                    start_half(i + nbuf // 2, h, buf, sem)

            return ()

        lax.fori_loop(0, main_halves // nbuf, loop_body, ())

        acc = acc_init
        for s in range(main_halves, total_halves):
            i, h = s // 2, s % 2
            buf, sem = bufs[s % nbuf]
            wait_half(i, h, buf, sem)
            acc = compute_half(buf, acc_init if h == 0 else acc)
            if h == 1:
                store(i, acc)
        pltpu.sync_copy(out_v, out_hbm.at[pl.ds(base, bpw)])

    return k


def kernel(input, table):
    B, L = input.shape
    V, D = table.shape
    idx3 = input.astype(jnp.int32).reshape(B, 2, L // 2)
    return _make_sc_kernel(V, D, B, L)(table, idx3)


# speedup vs baseline: 1.0013x; 1.0013x over previous
import functools

import jax
import jax.numpy as jnp
from jax import lax
from jax.experimental import pallas as pl
from jax.experimental.pallas import tpu as pltpu
from jax.experimental.pallas import tpu_sc as plsc

_INFO = plsc.get_sparse_core_info()
_NC = _INFO.num_cores
_NS = _INFO.num_subcores
_NW = _NC * _NS
_LANES = _INFO.num_lanes


def _make_sc_kernel(V, D, B, L):
    assert L % 2 == 0
    half = L // 2
    assert B % (8 * _NW) == 0 and D % _LANES == 0
    bpw = B // _NW
    ncol = D // _LANES

    mesh = plsc.VectorSubcoreMesh(core_axis_name="c", subcore_axis_name="s")

    @functools.partial(
        pl.kernel,
        mesh=mesh,
        out_type=jax.ShapeDtypeStruct((B, D), jnp.float32),
        scratch_types=[
            pltpu.VMEM((bpw, 2, half), jnp.int32),
            pltpu.VMEM((half, D), jnp.float32),
            pltpu.VMEM((half, D), jnp.float32),
            pltpu.VMEM((half, D), jnp.float32),
            pltpu.VMEM((half, D), jnp.float32),
            pltpu.VMEM((half, D), jnp.float32),
            pltpu.VMEM((half, D), jnp.float32),
            pltpu.VMEM((bpw, D), jnp.float32),
            pltpu.SemaphoreType.DMA,
            pltpu.SemaphoreType.DMA,
            pltpu.SemaphoreType.DMA,
            pltpu.SemaphoreType.DMA,
            pltpu.SemaphoreType.DMA,
            pltpu.SemaphoreType.DMA,
            pltpu.SemaphoreType.DMA,
        ],
    )
    def k(table_hbm, idx_hbm, out_hbm, idx_v,
          buf0, buf1, buf2, buf3, buf4, buf5, out_v,
          sem0, sem1, sem2, sem3, sem4, sem5, sem_idx):
        wid = lax.axis_index("s") * _NC + lax.axis_index("c")
        base = wid * bpw

        bufs = ((buf0, sem0), (buf1, sem1), (buf2, sem2),
                (buf3, sem3), (buf4, sem4), (buf5, sem5))
        nbuf = len(bufs)
        nprime = nbuf // 2

        pltpu.sync_copy(idx_hbm.at[pl.ds(base, nprime)],
                        idx_v.at[pl.ds(0, nprime)])
        idx_rest = pltpu.make_async_copy(
            idx_hbm.at[pl.ds(base + nprime, bpw - nprime)],
            idx_v.at[pl.ds(nprime, bpw - nprime)],
            sem_idx,
        )
        idx_rest.start()

        def start_half(i, h, buf, sem):
            pltpu.async_copy(table_hbm.at[idx_v.at[i, h]], buf, sem)

        def wait_half(i, h, buf, sem):
            pltpu.make_async_copy(
                table_hbm.at[idx_v.at[i, h]], buf, sem
            ).wait()

        UNROLL = 4
        assert half % UNROLL == 0

        acc_init = tuple(
            jnp.full((_LANES,), -jnp.inf, jnp.float32) for _ in range(ncol)
        )

        def compute_half(buf, acc_in):
            def row_body(jb, acc):
                j = jb * UNROLL
                out = []
                for c in range(ncol):
                    s = pl.ds(c * _LANES, _LANES)
                    m = [buf[j + u, s] for u in range(UNROLL)]
                    while len(m) > 1:
                        m = [jnp.maximum(m[2 * t], m[2 * t + 1])
                             for t in range(len(m) // 2)] + m[len(m) - len(m) % 2:]
                    out.append(jnp.maximum(acc[c], m[0]))
                return tuple(out)

            return lax.fori_loop(0, half // UNROLL, row_body, acc_in)

        def store(i, acc):
            for c in range(ncol):
                out_v[i, pl.ds(c * _LANES, _LANES)] = acc[c]

        for s in range(nbuf):
            start_half(s // 2, s % 2, bufs[s][0], bufs[s][1])
        idx_rest.wait()

        total_halves = 2 * bpw
        main_halves = (total_halves // nbuf) * nbuf
        assert nbuf % 2 == 0

        def loop_body(step, _):
            ebase = step * (nbuf // 2)
            acc = acc_init
            for b in range(nbuf):
                i = ebase + b // 2
                h = b % 2
                buf, sem = bufs[b]
                wait_half(i, h, buf, sem)
                acc = compute_half(buf, acc_init if h == 0 else acc)
                if h == 1:
                    store(i, acc)

                @pl.when(i + nbuf // 2 < bpw)
                def _():
                    start_half(i + nbuf // 2, h, buf, sem)

            return ()

        lax.fori_loop(0, main_halves // nbuf, loop_body, ())

        acc = acc_init
        for s in range(main_halves, total_halves):
            i, h = s // 2, s % 2
            buf, sem = bufs[s % nbuf]
            wait_half(i, h, buf, sem)
            acc = compute_half(buf, acc_init if h == 0 else acc)
            if h == 1:
                store(i, acc)
        pltpu.sync_copy(out_v, out_hbm.at[pl.ds(base, bpw)])

    return k


def kernel(input, table):
    B, L = input.shape
    V, D = table.shape
    idx3 = input.astype(jnp.int32).reshape(B, 2, L // 2)
    return _make_sc_kernel(V, D, B, L)(table, idx3)
